# baseline (speedup 1.0000x reference)
import functools
import jax
import jax.numpy as jnp
from jax.experimental import pallas as pl


def _uv_body(x_ref, a_ref, b_ref, bias_ref, u_ref, v_ref):
    x = x_ref[...]
    u_ref[...] = jnp.dot(x, a_ref[...], preferred_element_type=jnp.float32) + bias_ref[...]
    v_ref[...] = jnp.dot(x, b_ref[...], preferred_element_type=jnp.float32)


def _uv_matmul(xp, A, B, b1, blk):
    NP, f = xp.shape
    d_h = A.shape[1]
    grid = NP // blk
    return pl.pallas_call(
        _uv_body,
        grid=(grid,),
        in_specs=[
            pl.BlockSpec((blk, f), lambda i: (i, 0)),
            pl.BlockSpec((f, d_h), lambda i: (0, 0)),
            pl.BlockSpec((f, d_h), lambda i: (0, 0)),
            pl.BlockSpec((1, d_h), lambda i: (0, 0)),
        ],
        out_specs=[
            pl.BlockSpec((blk, d_h), lambda i: (i, 0)),
            pl.BlockSpec((blk, d_h), lambda i: (i, 0)),
        ],
        out_shape=[
            jax.ShapeDtypeStruct((NP, d_h), jnp.float32),
            jax.ShapeDtypeStruct((NP, d_h), jnp.float32),
        ],
    )(xp, A, B, b1.reshape(1, d_h))


def _edge_body(u_ref, v_ref, w2_ref, b2_ref, h_ref):
    m = jax.nn.relu(u_ref[...] + v_ref[...])
    h_ref[...] = jnp.dot(m, w2_ref[...], preferred_element_type=jnp.float32) + b2_ref[...]


def _edge_matmul(Ug, Vg, W2, b2, blk):
    E, d_h = Ug.shape
    d_out = W2.shape[1]
    grid = E // blk
    return pl.pallas_call(
        _edge_body,
        grid=(grid,),
        in_specs=[
            pl.BlockSpec((blk, d_h), lambda i: (i, 0)),
            pl.BlockSpec((blk, d_h), lambda i: (i, 0)),
            pl.BlockSpec((d_h, d_out), lambda i: (0, 0)),
            pl.BlockSpec((1, d_out), lambda i: (0, 0)),
        ],
        out_specs=pl.BlockSpec((blk, d_out), lambda i: (i, 0)),
        out_shape=jax.ShapeDtypeStruct((E, d_out), jnp.float32),
    )(Ug, Vg, W2, b2.reshape(1, d_out))


NP = 10240


def _layer(xp, src, dst, W1, b1, W2, b2):
    f = xp.shape[1]
    A = W1[:f] - W1[f:]
    B = W1[f:]
    U, V = _uv_matmul(xp, A, B, b1, blk=512)
    Ug = jnp.take(U, dst, axis=0)
    Vg = jnp.take(V, src, axis=0)
    H = _edge_matmul(Ug, Vg, W2, b2, blk=512)
    agg = jax.ops.segment_max(H, dst, num_segments=NP)
    return jnp.maximum(jnp.where(jnp.isfinite(agg), agg, 0.0), 0.0)


@jax.jit
def kernel(x, edge_index, W1_1, b1_1, W2_1, b2_1, W1_2, b1_2, W2_2, b2_2,
           W1_3, b1_3, W2_3, b2_3, W1_4, b1_4, W2_4, b2_4):
    N = x.shape[0]
    src = edge_index[0]
    dst = edge_index[1]
    xp = jnp.pad(x, ((0, NP - N), (0, 0)))
    xp = _layer(xp, src, dst, W1_1, b1_1, W2_1, b2_1)
    xp = _layer(xp, src, dst, W1_2, b1_2, W2_2, b2_2)
    xp = _layer(xp, src, dst, W1_3, b1_3, W2_3, b2_3)
    xp = _layer(xp, src, dst, W1_4, b1_4, W2_4, b2_4)
    return xp[:N]


# speedup vs baseline: 1.7898x; 1.7898x over previous
import functools
import jax
import jax.numpy as jnp
from jax import lax
from jax.experimental import pallas as pl
from jax.experimental.pallas import tpu as pltpu
from jax.experimental.pallas import tpu_sc as plsc

_N = 10000
_E = 320000
_NW = 32
_BKT = 320
_NP = _NW * _BKT
_CH = 6400
_FLUSH = 2048
_CAP = 322048
_B = 128

_mesh = functools.partial(
    plsc.VectorSubcoreMesh, core_axis_name="c", subcore_axis_name="s")


def _wid():
    return lax.axis_index("s") * 2 + lax.axis_index("c")


def _uv_body(x_ref, a_ref, b_ref, bias_ref, u_ref, v_ref):
    x = x_ref[...]
    u_ref[...] = jnp.dot(x, a_ref[...], preferred_element_type=jnp.float32) + bias_ref[...]
    v_ref[...] = jnp.dot(x, b_ref[...], preferred_element_type=jnp.float32)


def _uv_matmul(xp, A, B, b1, blk=512):
    NP, f = xp.shape
    d_h = A.shape[1]
    return pl.pallas_call(
        _uv_body,
        grid=(NP // blk,),
        in_specs=[
            pl.BlockSpec((blk, f), lambda i: (i, 0)),
            pl.BlockSpec((f, d_h), lambda i: (0, 0)),
            pl.BlockSpec((f, d_h), lambda i: (0, 0)),
            pl.BlockSpec((1, d_h), lambda i: (0, 0)),
        ],
        out_specs=[
            pl.BlockSpec((blk, d_h), lambda i: (i, 0)),
            pl.BlockSpec((blk, d_h), lambda i: (i, 0)),
        ],
        out_shape=[
            jax.ShapeDtypeStruct((NP, d_h), jnp.float32),
            jax.ShapeDtypeStruct((NP, d_h), jnp.float32),
        ],
    )(xp, A, B, b1.reshape(1, d_h))


def _edge_body(u_ref, v_ref, w2_ref, b2_ref, h_ref):
    m = jax.nn.relu(u_ref[...] + v_ref[...])
    h_ref[...] = jnp.dot(m, w2_ref[...], preferred_element_type=jnp.float32) + b2_ref[...]


def _edge_matmul(Ug, Vg, W2, b2, blk=512):
    E, d_h = Ug.shape
    d_out = W2.shape[1]
    return pl.pallas_call(
        _edge_body,
        grid=(E // blk,),
        in_specs=[
            pl.BlockSpec((blk, d_h), lambda i: (i, 0)),
            pl.BlockSpec((blk, d_h), lambda i: (i, 0)),
            pl.BlockSpec((d_h, d_out), lambda i: (0, 0)),
            pl.BlockSpec((1, d_out), lambda i: (0, 0)),
        ],
        out_specs=pl.BlockSpec((blk, d_out), lambda i: (i, 0)),
        out_shape=jax.ShapeDtypeStruct((E, d_out), jnp.float32),
    )(Ug, Vg, W2, b2.reshape(1, d_out))


def _prep_sc(dst):
    @functools.partial(
        pl.kernel,
        mesh=_mesh(),
        compiler_params=pltpu.CompilerParams(needs_layout_passes=False),
        out_type=[
            jax.ShapeDtypeStruct((_NW * _CAP,), jnp.int32),
            jax.ShapeDtypeStruct((_NW * 16,), jnp.int32),
        ],
        scratch_types=[
            pltpu.VMEM((_CH,), jnp.int32),
            pltpu.VMEM((_FLUSH + 16,), jnp.int32),
            pltpu.VMEM((16,), jnp.int32),
        ],
    )
    def prep(dst_hbm, pk_hbm, cnt_hbm, dchunk, pkbuf, stg):
        wid = _wid()
        lo = wid * _BKT
        iota = lax.iota(jnp.int32, 16)
        sent = jnp.full((16,), _BKT, jnp.int32)

        def flush(off, base):
            pltpu.sync_copy(pkbuf.at[pl.ds(0, _FLUSH)],
                            pk_hbm.at[pl.ds(pl.multiple_of(wid * _CAP + base, _FLUSH), _FLUSH)])
            pkbuf[pl.ds(0, 16)] = pkbuf[pl.ds(_FLUSH, 16)]
            return off - _FLUSH, base + _FLUSH

        def maybe_flush(off, base):
            return lax.cond(off >= _FLUSH, flush, lambda o, b: (o, b), off, base)

        def chunk_body(ci, carry):
            pltpu.sync_copy(dst_hbm.at[pl.ds(pl.multiple_of(ci * _CH, _CH), _CH)], dchunk)

            def vec_body(v, carry):
                off, base = carry
                d16 = dchunk[pl.ds(v * 16, 16)]
                m = (d16 >= lo) & (d16 < lo + _BKT)
                packed = (ci * _CH + v * 16 + iota) * 512 + (d16 - lo)
                pk_sorted = lax.sort(
                    jnp.where(m, packed, jnp.int32(0x7FFFFFFF)), dimension=0)
                pkbuf[pl.ds(off, 16)] = pk_sorted
                off = off + jnp.sum(m.astype(jnp.int32))
                return maybe_flush(off, base)

            return lax.fori_loop(0, _CH // 16, vec_body, carry)

        off, base = lax.fori_loop(0, _E // _CH, chunk_body, (0, 0))

        pad = (-off) % _B

        def pad_body(i, carry):
            off, base = carry

            def do(off, base):
                pkbuf[pl.ds(off, 16)] = sent
                return maybe_flush(off + 16, base)

            return lax.cond(i * 16 < pad, do, lambda o, b: (o, b), off, base)

        off, base = lax.fori_loop(0, _B // 16, pad_body, (off, base))
        pltpu.sync_copy(pkbuf.at[pl.ds(0, _FLUSH)],
                        pk_hbm.at[pl.ds(pl.multiple_of(wid * _CAP + base, _FLUSH), _FLUSH)])
        stg[pl.ds(0, 16)] = jnp.full((16,), 1, jnp.int32) * (base + off)
        pltpu.sync_copy(stg.at[pl.ds(0, 16)], cnt_hbm.at[pl.ds(pl.multiple_of(wid * 16, 16), 16)])

    return prep(dst)


def _gather_uv(U, V, src, dst):
    d_h = U.shape[1]
    nb_all = _E // _B
    per = nb_all // _NW
    rem = nb_all - per * _NW

    @functools.partial(
        pl.kernel,
        mesh=_mesh(),
        out_type=[
            jax.ShapeDtypeStruct((_E, d_h), jnp.float32),
            jax.ShapeDtypeStruct((_E, d_h), jnp.float32),
        ],
        scratch_types=[
            pltpu.VMEM((_B,), jnp.int32),
            pltpu.VMEM((_B,), jnp.int32),
            pltpu.VMEM((_B, d_h), jnp.float32),
            pltpu.VMEM((_B, d_h), jnp.float32),
            pltpu.SemaphoreType.DMA,
            pltpu.SemaphoreType.DMA,
        ],
    )
    def gat(u_hbm, v_hbm, src_hbm, dst_hbm, ug_hbm, vg_hbm,
            idxd, idxs, ubuf, vbuf, semu, semv):
        wid = _wid()
        start = wid * per + jnp.minimum(wid, rem)
        nb = jnp.where(wid < rem, per + 1, per)

        def body(k, _):
            e0 = pl.multiple_of((start + k) * _B, _B)
            pltpu.sync_copy(dst_hbm.at[pl.ds(e0, _B)], idxd)
            pltpu.sync_copy(src_hbm.at[pl.ds(e0, _B)], idxs)
            cu = pltpu.async_copy(u_hbm.at[idxd], ubuf, semu)
            cv = pltpu.async_copy(v_hbm.at[idxs], vbuf, semv)
            cu.wait()
            cv.wait()
            pltpu.sync_copy(ubuf, ug_hbm.at[pl.ds(e0, _B)])
            pltpu.sync_copy(vbuf, vg_hbm.at[pl.ds(e0, _B)])
            return 0

        lax.fori_loop(0, nb, body, 0)

    return gat(U, V, src, dst)


def _scatter_max(H, pk, cnts):
    d_out = H.shape[1]

    @functools.partial(
        pl.kernel,
        mesh=_mesh(),
        out_type=jax.ShapeDtypeStruct((_NP, d_out), jnp.float32),
        scratch_types=[
            pltpu.VMEM((_BKT + 1, d_out), jnp.float32),
            pltpu.VMEM((_B, d_out), jnp.float32),
            pltpu.VMEM((_B + 16,), jnp.int32),
            pltpu.VMEM((_B,), jnp.int32),
            pltpu.VMEM((16,), jnp.int32),
            pltpu.SemaphoreType.DMA,
        ],
    )
    def scat(h_hbm, pk_hbm, cnt_hbm, out_hbm,
             tbl, hbuf, pkv, idxb, cntv, sem):
        wid = _wid()
        zero16 = jnp.zeros((16,), jnp.float32)

        def zb(i, _):
            for j in range(d_out // 16):
                tbl[i, pl.ds(j * 16, 16)] = zero16
            return 0

        lax.fori_loop(0, _BKT + 1, zb, 0)

        pltpu.sync_copy(cnt_hbm.at[pl.ds(pl.multiple_of(wid * 16, 16), 16)], cntv)
        nb = cntv[pl.ds(0, 16)][0] // _B

        def bb(k, _):
            boff = pl.multiple_of(wid * _CAP + k * _B, _B)
            pltpu.sync_copy(pk_hbm.at[pl.ds(boff, _B)], pkv.at[pl.ds(0, _B)])
            for j in range(_B // 16):
                idxb[pl.ds(j * 16, 16)] = lax.shift_right_logical(
                    pkv[pl.ds(j * 16, 16)], 9)
            pltpu.async_copy(h_hbm.at[idxb], hbuf, sem).wait()

            def eb(i, _):
                loc = pkv[pl.ds(i, 16)][0] & 511
                for j in range(d_out // 16):
                    a = tbl[loc, pl.ds(j * 16, 16)]
                    b = hbuf[i, pl.ds(j * 16, 16)]
                    tbl[loc, pl.ds(j * 16, 16)] = jnp.maximum(a, b)
                return 0

            lax.fori_loop(0, _B, eb, 0)
            return 0

        lax.fori_loop(0, nb, bb, 0)
        pltpu.sync_copy(tbl.at[pl.ds(0, _BKT)],
                        out_hbm.at[pl.ds(pl.multiple_of(wid * _BKT, _BKT), _BKT)])

    return scat(H, pk, cnts)


def _layer(xp, src, dst, pk, cnts, W1, b1, W2, b2):
    f = xp.shape[1]
    A = W1[:f] - W1[f:]
    B = W1[f:]
    U, V = _uv_matmul(xp, A, B, b1)
    Ug, Vg = _gather_uv(U, V, src, dst)
    H = _edge_matmul(Ug, Vg, W2, b2)
    return _scatter_max(H, pk, cnts)


@jax.jit
def kernel(x, edge_index, W1_1, b1_1, W2_1, b2_1, W1_2, b1_2, W2_2, b2_2,
           W1_3, b1_3, W2_3, b2_3, W1_4, b1_4, W2_4, b2_4):
    N = x.shape[0]
    src = edge_index[0]
    dst = edge_index[1]
    pk, cnts = _prep_sc(dst)
    xp = jnp.pad(x, ((0, _NP - N), (0, 0)))
    xp = _layer(xp, src, dst, pk, cnts, W1_1, b1_1, W2_1, b2_1)
    xp = _layer(xp, src, dst, pk, cnts, W1_2, b1_2, W2_2, b2_2)
    xp = _layer(xp, src, dst, pk, cnts, W1_3, b1_3, W2_3, b2_3)
    xp = _layer(xp, src, dst, pk, cnts, W1_4, b1_4, W2_4, b2_4)
    return xp[:N]


# speedup vs baseline: 2.0391x; 1.1393x over previous
import functools
import jax
import jax.numpy as jnp
from jax import lax
from jax.experimental import pallas as pl
from jax.experimental.pallas import tpu as pltpu
from jax.experimental.pallas import tpu_sc as plsc

_N = 10000
_E = 320000
_NW = 32
_BKT = 320
_NP = _NW * _BKT
_CH = 6400
_FLUSH = 2048
_CAP = 322048
_B = 128

_mesh = functools.partial(
    plsc.VectorSubcoreMesh, core_axis_name="c", subcore_axis_name="s")


def _wid():
    return lax.axis_index("s") * 2 + lax.axis_index("c")


def _uv_body(x_ref, a_ref, b_ref, bias_ref, u_ref, v_ref):
    x = x_ref[...]
    u_ref[...] = jnp.dot(x, a_ref[...], preferred_element_type=jnp.float32) + bias_ref[...]
    v_ref[...] = jnp.dot(x, b_ref[...], preferred_element_type=jnp.float32)


def _uv_matmul(xp, A, B, b1, blk=512):
    NP, f = xp.shape
    d_h = A.shape[1]
    return pl.pallas_call(
        _uv_body,
        grid=(NP // blk,),
        in_specs=[
            pl.BlockSpec((blk, f), lambda i: (i, 0)),
            pl.BlockSpec((f, d_h), lambda i: (0, 0)),
            pl.BlockSpec((f, d_h), lambda i: (0, 0)),
            pl.BlockSpec((1, d_h), lambda i: (0, 0)),
        ],
        out_specs=[
            pl.BlockSpec((blk, d_h), lambda i: (i, 0)),
            pl.BlockSpec((blk, d_h), lambda i: (i, 0)),
        ],
        out_shape=[
            jax.ShapeDtypeStruct((NP, d_h), jnp.float32),
            jax.ShapeDtypeStruct((NP, d_h), jnp.float32),
        ],
    )(xp, A, B, b1.reshape(1, d_h))


def _edge_body(ms_ref, w2_ref, b2_ref, h_ref):
    m = jax.nn.relu(ms_ref[...])
    h_ref[...] = jnp.dot(m, w2_ref[...], preferred_element_type=jnp.float32) + b2_ref[...]


def _edge_matmul(ms, W2, b2, blk=512):
    E, d_h = ms.shape
    d_out = W2.shape[1]
    return pl.pallas_call(
        _edge_body,
        grid=(E // blk,),
        in_specs=[
            pl.BlockSpec((blk, d_h), lambda i: (i, 0)),
            pl.BlockSpec((d_h, d_out), lambda i: (0, 0)),
            pl.BlockSpec((1, d_out), lambda i: (0, 0)),
        ],
        out_specs=pl.BlockSpec((blk, d_out), lambda i: (i, 0)),
        out_shape=jax.ShapeDtypeStruct((E, d_out), jnp.float32),
    )(ms, W2, b2.reshape(1, d_out))


def _prep_sc(dst):
    @functools.partial(
        pl.kernel,
        mesh=_mesh(),
        compiler_params=pltpu.CompilerParams(needs_layout_passes=False),
        out_type=[
            jax.ShapeDtypeStruct((_NW * _CAP,), jnp.int32),
            jax.ShapeDtypeStruct((_NW * 16,), jnp.int32),
        ],
        scratch_types=[
            pltpu.VMEM((_CH,), jnp.int32),
            pltpu.VMEM((_FLUSH + 16,), jnp.int32),
            pltpu.VMEM((16,), jnp.int32),
        ],
    )
    def prep(dst_hbm, pk_hbm, cnt_hbm, dchunk, pkbuf, stg):
        wid = _wid()
        lo = wid * _BKT
        iota = lax.iota(jnp.int32, 16)
        sent = jnp.full((16,), _BKT, jnp.int32)

        def flush(off, base):
            pltpu.sync_copy(pkbuf.at[pl.ds(0, _FLUSH)],
                            pk_hbm.at[pl.ds(pl.multiple_of(wid * _CAP + base, _FLUSH), _FLUSH)])
            pkbuf[pl.ds(0, 16)] = pkbuf[pl.ds(_FLUSH, 16)]
            return off - _FLUSH, base + _FLUSH

        def maybe_flush(off, base):
            return lax.cond(off >= _FLUSH, flush, lambda o, b: (o, b), off, base)

        def chunk_body(ci, carry):
            pltpu.sync_copy(dst_hbm.at[pl.ds(pl.multiple_of(ci * _CH, _CH), _CH)], dchunk)

            def vec_body(v, carry):
                off, base = carry
                d16 = dchunk[pl.ds(v * 16, 16)]
                m = (d16 >= lo) & (d16 < lo + _BKT)
                packed = (ci * _CH + v * 16 + iota) * 512 + (d16 - lo)
                pk_sorted = lax.sort(
                    jnp.where(m, packed, jnp.int32(0x7FFFFFFF)), dimension=0)
                pkbuf[pl.ds(off, 16)] = pk_sorted
                off = off + jnp.sum(m.astype(jnp.int32))
                return maybe_flush(off, base)

            return lax.fori_loop(0, _CH // 16, vec_body, carry)

        off, base = lax.fori_loop(0, _E // _CH, chunk_body, (0, 0))

        pad = (-off) % _B

        def pad_body(i, carry):
            off, base = carry

            def do(off, base):
                pkbuf[pl.ds(off, 16)] = sent
                return maybe_flush(off + 16, base)

            return lax.cond(i * 16 < pad, do, lambda o, b: (o, b), off, base)

        off, base = lax.fori_loop(0, _B // 16, pad_body, (off, base))
        pltpu.sync_copy(pkbuf.at[pl.ds(0, _FLUSH)],
                        pk_hbm.at[pl.ds(pl.multiple_of(wid * _CAP + base, _FLUSH), _FLUSH)])
        stg[pl.ds(0, 16)] = jnp.full((16,), 1, jnp.int32) * (base + off)
        pltpu.sync_copy(stg.at[pl.ds(0, 16)], cnt_hbm.at[pl.ds(pl.multiple_of(wid * 16, 16), 16)])

    return prep(dst)


_BG = 64
_NBG = (_E // _BG) // _NW
_RMG = _E // _BG - _NBG * _NW

def _gather_msum(U, V, src, dst):
    d_h = U.shape[1]

    @functools.partial(
        pl.kernel,
        mesh=_mesh(),
        out_type=jax.ShapeDtypeStruct((_E, d_h), jnp.float32),
        scratch_types=[
            pltpu.VMEM((_NBG * _BG + _BG,), jnp.int32),
            pltpu.VMEM((_NBG * _BG + _BG,), jnp.int32),
            pltpu.VMEM((3, _BG, d_h), jnp.float32),
            pltpu.VMEM((3, _BG, d_h), jnp.float32),
        ] + [pltpu.SemaphoreType.DMA] * 9,
    )
    def gat(u_hbm, v_hbm, src_hbm, dst_hbm, ms_hbm,
            idxd, idxs, ub, vb, gu0, gu1, gu2, gv0, gv1, gv2, s0, s1, s2):
        wid = _wid()
        gu = [gu0, gu1, gu2]
        gv = [gv0, gv1, gv2]
        ss = [s0, s1, s2]
        start = wid * _NBG
        e_lo = pl.multiple_of(start * _BG, _BG)
        pltpu.sync_copy(dst_hbm.at[pl.ds(e_lo, _NBG * _BG)],
                        idxd.at[pl.ds(0, _NBG * _BG)])
        pltpu.sync_copy(src_hbm.at[pl.ds(e_lo, _NBG * _BG)],
                        idxs.at[pl.ds(0, _NBG * _BG)])

        def uv_start(k, slot):
            pltpu.async_copy(
                u_hbm.at[idxd.at[pl.ds(k * _BG, _BG)]], ub.at[slot], gu[slot])
            pltpu.async_copy(
                v_hbm.at[idxs.at[pl.ds(k * _BG, _BG)]], vb.at[slot], gv[slot])

        def uv_wait(k, slot):
            pltpu.make_async_copy(
                u_hbm.at[idxd.at[pl.ds(k * _BG, _BG)]], ub.at[slot],
                gu[slot]).wait()
            pltpu.make_async_copy(
                v_hbm.at[idxs.at[pl.ds(k * _BG, _BG)]], vb.at[slot],
                gv[slot]).wait()

        def add_uv(slot):
            def ab(i, _):
                for j in range(d_h // 16):
                    ub[slot, i, pl.ds(j * 16, 16)] = (
                        ub[slot, i, pl.ds(j * 16, 16)]
                        + vb[slot, i, pl.ds(j * 16, 16)])
                return 0

            lax.fori_loop(0, _BG, ab, 0)

        def st_wait(k, slot):
            pltpu.make_async_copy(
                ub.at[slot], ms_hbm.at[pl.ds((start + k) * _BG, _BG)],
                ss[slot]).wait()

        uv_start(0, 0)

        def body(kk, _):
            for j in range(3):
                k = kk * 3 + j
                nslot = (j + 1) % 3

                @pl.when(k >= 2)
                def _():
                    st_wait(k - 2, nslot)

                @pl.when(k + 1 < _NBG)
                def _():
                    uv_start(k + 1, nslot)

                uv_wait(k, j)
                add_uv(j)
                pltpu.async_copy(
                    ub.at[j], ms_hbm.at[pl.ds((start + k) * _BG, _BG)], ss[j])
            return 0

        lax.fori_loop(0, _NBG // 3, body, 0)
        st_wait(_NBG - 2, (_NBG - 2) % 3)
        st_wait(_NBG - 1, (_NBG - 1) % 3)

        @pl.when(wid < _RMG)
        def _():
            e0 = pl.multiple_of((_NW * _NBG + wid) * _BG, _BG)
            pltpu.sync_copy(dst_hbm.at[pl.ds(e0, _BG)],
                            idxd.at[pl.ds(_NBG * _BG, _BG)])
            pltpu.sync_copy(src_hbm.at[pl.ds(e0, _BG)],
                            idxs.at[pl.ds(_NBG * _BG, _BG)])
            pltpu.async_copy(
                u_hbm.at[idxd.at[pl.ds(_NBG * _BG, _BG)]], ub.at[0],
                gu[0]).wait()
            pltpu.async_copy(
                v_hbm.at[idxs.at[pl.ds(_NBG * _BG, _BG)]], vb.at[0],
                gv[0]).wait()
            add_uv(0)
            pltpu.sync_copy(ub.at[0], ms_hbm.at[pl.ds(e0, _BG)])

    return gat(U, V, src, dst)


def _scatter_max(H, pk, cnts):
    d_out = H.shape[1]

    @functools.partial(
        pl.kernel,
        mesh=_mesh(),
        out_type=jax.ShapeDtypeStruct((_NP, d_out), jnp.float32),
        scratch_types=[
            pltpu.VMEM((_BKT + 1, d_out), jnp.float32),
            pltpu.VMEM((_B, d_out), jnp.float32),
            pltpu.VMEM((_B + 16,), jnp.int32),
            pltpu.VMEM((_B,), jnp.int32),
            pltpu.VMEM((16,), jnp.int32),
            pltpu.SemaphoreType.DMA,
        ],
    )
    def scat(h_hbm, pk_hbm, cnt_hbm, out_hbm,
             tbl, hbuf, pkv, idxb, cntv, sem):
        wid = _wid()
        zero16 = jnp.zeros((16,), jnp.float32)

        def zb(i, _):
            for j in range(d_out // 16):
                tbl[i, pl.ds(j * 16, 16)] = zero16
            return 0

        lax.fori_loop(0, _BKT + 1, zb, 0)

        pltpu.sync_copy(cnt_hbm.at[pl.ds(pl.multiple_of(wid * 16, 16), 16)], cntv)
        nb = cntv[pl.ds(0, 16)][0] // _B

        def bb(k, _):
            boff = pl.multiple_of(wid * _CAP + k * _B, _B)
            pltpu.sync_copy(pk_hbm.at[pl.ds(boff, _B)], pkv.at[pl.ds(0, _B)])
            for j in range(_B // 16):
                idxb[pl.ds(j * 16, 16)] = lax.shift_right_logical(
                    pkv[pl.ds(j * 16, 16)], 9)
            pltpu.async_copy(h_hbm.at[idxb], hbuf, sem).wait()

            def eb(i, _):
                loc = pkv[pl.ds(i, 16)][0] & 511
                for j in range(d_out // 16):
                    a = tbl[loc, pl.ds(j * 16, 16)]
                    b = hbuf[i, pl.ds(j * 16, 16)]
                    tbl[loc, pl.ds(j * 16, 16)] = jnp.maximum(a, b)
                return 0

            lax.fori_loop(0, _B, eb, 0)
            return 0

        lax.fori_loop(0, nb, bb, 0)
        pltpu.sync_copy(tbl.at[pl.ds(0, _BKT)],
                        out_hbm.at[pl.ds(pl.multiple_of(wid * _BKT, _BKT), _BKT)])

    return scat(H, pk, cnts)


def _layer(xp, src, dst, pk, cnts, W1, b1, W2, b2):
    f = xp.shape[1]
    A = W1[:f] - W1[f:]
    B = W1[f:]
    U, V = _uv_matmul(xp, A, B, b1)
    ms = _gather_msum(U, V, src, dst)
    H = _edge_matmul(ms, W2, b2)
    return _scatter_max(H, pk, cnts)


@jax.jit
def kernel(x, edge_index, W1_1, b1_1, W2_1, b2_1, W1_2, b1_2, W2_2, b2_2,
           W1_3, b1_3, W2_3, b2_3, W1_4, b1_4, W2_4, b2_4):
    N = x.shape[0]
    src = edge_index[0]
    dst = edge_index[1]
    pk, cnts = _prep_sc(dst)
    xp = jnp.pad(x, ((0, _NP - N), (0, 0)))
    xp = _layer(xp, src, dst, pk, cnts, W1_1, b1_1, W2_1, b2_1)
    xp = _layer(xp, src, dst, pk, cnts, W1_2, b1_2, W2_2, b2_2)
    xp = _layer(xp, src, dst, pk, cnts, W1_3, b1_3, W2_3, b2_3)
    xp = _layer(xp, src, dst, pk, cnts, W1_4, b1_4, W2_4, b2_4)
    return xp[:N]


# speedup vs baseline: 2.2172x; 1.0873x over previous
import functools
import jax
import jax.numpy as jnp
from jax import lax
from jax.experimental import pallas as pl
from jax.experimental.pallas import tpu as pltpu
from jax.experimental.pallas import tpu_sc as plsc

_N = 10000
_E = 320000
_NW = 32
_BKT = 320
_NP = _NW * _BKT
_CH = 6400
_FLUSH = 2048
_CAP = 322048
_B = 128

_mesh = functools.partial(
    plsc.VectorSubcoreMesh, core_axis_name="c", subcore_axis_name="s")


def _wid():
    return lax.axis_index("s") * 2 + lax.axis_index("c")


def _uv_body(x_ref, a_ref, b_ref, bias_ref, u_ref, v_ref):
    x = x_ref[...]
    u_ref[...] = jnp.dot(x, a_ref[...], preferred_element_type=jnp.float32) + bias_ref[...]
    v_ref[...] = jnp.dot(x, b_ref[...], preferred_element_type=jnp.float32)


def _uv_matmul(xp, A, B, b1, blk=512):
    NP, f = xp.shape
    d_h = A.shape[1]
    return pl.pallas_call(
        _uv_body,
        grid=(NP // blk,),
        in_specs=[
            pl.BlockSpec((blk, f), lambda i: (i, 0)),
            pl.BlockSpec((f, d_h), lambda i: (0, 0)),
            pl.BlockSpec((f, d_h), lambda i: (0, 0)),
            pl.BlockSpec((1, d_h), lambda i: (0, 0)),
        ],
        out_specs=[
            pl.BlockSpec((blk, d_h), lambda i: (i, 0)),
            pl.BlockSpec((blk, d_h), lambda i: (i, 0)),
        ],
        out_shape=[
            jax.ShapeDtypeStruct((NP, d_h), jnp.float32),
            jax.ShapeDtypeStruct((NP, d_h), jnp.float32),
        ],
    )(xp, A, B, b1.reshape(1, d_h))


def _edge_body(ms_ref, w2_ref, b2_ref, h_ref):
    m = jax.nn.relu(ms_ref[...])
    h_ref[...] = jnp.dot(m, w2_ref[...], preferred_element_type=jnp.float32) + b2_ref[...]


def _edge_matmul(ms, W2, b2, blk=512):
    E, d_h = ms.shape
    d_out = W2.shape[1]
    return pl.pallas_call(
        _edge_body,
        grid=(E // blk,),
        in_specs=[
            pl.BlockSpec((blk, d_h), lambda i: (i, 0)),
            pl.BlockSpec((d_h, d_out), lambda i: (0, 0)),
            pl.BlockSpec((1, d_out), lambda i: (0, 0)),
        ],
        out_specs=pl.BlockSpec((blk, d_out), lambda i: (i, 0)),
        out_shape=jax.ShapeDtypeStruct((E, d_out), jnp.float32),
    )(ms, W2, b2.reshape(1, d_out))


def _prep_sc(dst):
    @functools.partial(
        pl.kernel,
        mesh=_mesh(),
        compiler_params=pltpu.CompilerParams(needs_layout_passes=False),
        out_type=[
            jax.ShapeDtypeStruct((_NW * _CAP,), jnp.int32),
            jax.ShapeDtypeStruct((_NW * 16,), jnp.int32),
        ],
        scratch_types=[
            pltpu.VMEM((_CH,), jnp.int32),
            pltpu.VMEM((_FLUSH + 16,), jnp.int32),
            pltpu.VMEM((16,), jnp.int32),
        ],
    )
    def prep(dst_hbm, pk_hbm, cnt_hbm, dchunk, pkbuf, stg):
        wid = _wid()
        lo = wid * _BKT
        iota = lax.iota(jnp.int32, 16)
        sent = jnp.full((16,), _BKT, jnp.int32)

        def flush(off, base):
            pltpu.sync_copy(pkbuf.at[pl.ds(0, _FLUSH)],
                            pk_hbm.at[pl.ds(pl.multiple_of(wid * _CAP + base, _FLUSH), _FLUSH)])
            pkbuf[pl.ds(0, 16)] = pkbuf[pl.ds(_FLUSH, 16)]
            return off - _FLUSH, base + _FLUSH

        def maybe_flush(off, base):
            return lax.cond(off >= _FLUSH, flush, lambda o, b: (o, b), off, base)

        def chunk_body(ci, carry):
            pltpu.sync_copy(dst_hbm.at[pl.ds(pl.multiple_of(ci * _CH, _CH), _CH)], dchunk)

            def vec_body(v, carry):
                off, base = carry
                d16 = dchunk[pl.ds(v * 16, 16)]
                m = (d16 >= lo) & (d16 < lo + _BKT)
                packed = (ci * _CH + v * 16 + iota) * 512 + (d16 - lo)
                pk_sorted = lax.sort(
                    jnp.where(m, packed, jnp.int32(0x7FFFFFFF)), dimension=0)
                pkbuf[pl.ds(off, 16)] = pk_sorted
                off = off + jnp.sum(m.astype(jnp.int32))
                return maybe_flush(off, base)

            return lax.fori_loop(0, _CH // 16, vec_body, carry)

        off, base = lax.fori_loop(0, _E // _CH, chunk_body, (0, 0))

        pad = (-off) % _B

        def pad_body(i, carry):
            off, base = carry

            def do(off, base):
                pkbuf[pl.ds(off, 16)] = sent
                return maybe_flush(off + 16, base)

            return lax.cond(i * 16 < pad, do, lambda o, b: (o, b), off, base)

        off, base = lax.fori_loop(0, _B // 16, pad_body, (off, base))
        pltpu.sync_copy(pkbuf.at[pl.ds(0, _FLUSH)],
                        pk_hbm.at[pl.ds(pl.multiple_of(wid * _CAP + base, _FLUSH), _FLUSH)])
        stg[pl.ds(0, 16)] = jnp.full((16,), 1, jnp.int32) * (base + off)
        pltpu.sync_copy(stg.at[pl.ds(0, 16)], cnt_hbm.at[pl.ds(pl.multiple_of(wid * 16, 16), 16)])

    return prep(dst)


_BG = 64
_NBG = (_E // _BG) // _NW
_RMG = _E // _BG - _NBG * _NW

def _gather_msum(U, V, src, dst):
    d_h = U.shape[1]

    @functools.partial(
        pl.kernel,
        mesh=_mesh(),
        out_type=jax.ShapeDtypeStruct((_E, d_h), jnp.float32),
        scratch_types=[
            pltpu.VMEM((_NBG * _BG + _BG,), jnp.int32),
            pltpu.VMEM((_NBG * _BG + _BG,), jnp.int32),
            pltpu.VMEM((3, _BG, d_h), jnp.float32),
            pltpu.VMEM((3, _BG, d_h), jnp.float32),
        ] + [pltpu.SemaphoreType.DMA] * 9,
    )
    def gat(u_hbm, v_hbm, src_hbm, dst_hbm, ms_hbm,
            idxd, idxs, ub, vb, gu0, gu1, gu2, gv0, gv1, gv2, s0, s1, s2):
        wid = _wid()
        gu = [gu0, gu1, gu2]
        gv = [gv0, gv1, gv2]
        ss = [s0, s1, s2]
        start = wid * _NBG
        e_lo = pl.multiple_of(start * _BG, _BG)
        pltpu.sync_copy(dst_hbm.at[pl.ds(e_lo, _NBG * _BG)],
                        idxd.at[pl.ds(0, _NBG * _BG)])
        pltpu.sync_copy(src_hbm.at[pl.ds(e_lo, _NBG * _BG)],
                        idxs.at[pl.ds(0, _NBG * _BG)])

        def uv_start(k, slot):
            pltpu.async_copy(
                u_hbm.at[idxd.at[pl.ds(k * _BG, _BG)]], ub.at[slot], gu[slot])
            pltpu.async_copy(
                v_hbm.at[idxs.at[pl.ds(k * _BG, _BG)]], vb.at[slot], gv[slot])

        def uv_wait(k, slot):
            pltpu.make_async_copy(
                u_hbm.at[idxd.at[pl.ds(k * _BG, _BG)]], ub.at[slot],
                gu[slot]).wait()
            pltpu.make_async_copy(
                v_hbm.at[idxs.at[pl.ds(k * _BG, _BG)]], vb.at[slot],
                gv[slot]).wait()

        def add_uv(slot):
            def ab(i, _):
                for j in range(d_h // 16):
                    ub[slot, i, pl.ds(j * 16, 16)] = (
                        ub[slot, i, pl.ds(j * 16, 16)]
                        + vb[slot, i, pl.ds(j * 16, 16)])
                return 0

            lax.fori_loop(0, _BG, ab, 0)

        def st_wait(k, slot):
            pltpu.make_async_copy(
                ub.at[slot], ms_hbm.at[pl.ds((start + k) * _BG, _BG)],
                ss[slot]).wait()

        uv_start(0, 0)

        def body(kk, _):
            for j in range(3):
                k = kk * 3 + j
                nslot = (j + 1) % 3

                @pl.when(k >= 2)
                def _():
                    st_wait(k - 2, nslot)

                @pl.when(k + 1 < _NBG)
                def _():
                    uv_start(k + 1, nslot)

                uv_wait(k, j)
                add_uv(j)
                pltpu.async_copy(
                    ub.at[j], ms_hbm.at[pl.ds((start + k) * _BG, _BG)], ss[j])
            return 0

        lax.fori_loop(0, _NBG // 3, body, 0)
        st_wait(_NBG - 2, (_NBG - 2) % 3)
        st_wait(_NBG - 1, (_NBG - 1) % 3)

        @pl.when(wid < _RMG)
        def _():
            e0 = pl.multiple_of((_NW * _NBG + wid) * _BG, _BG)
            pltpu.sync_copy(dst_hbm.at[pl.ds(e0, _BG)],
                            idxd.at[pl.ds(_NBG * _BG, _BG)])
            pltpu.sync_copy(src_hbm.at[pl.ds(e0, _BG)],
                            idxs.at[pl.ds(_NBG * _BG, _BG)])
            pltpu.async_copy(
                u_hbm.at[idxd.at[pl.ds(_NBG * _BG, _BG)]], ub.at[0],
                gu[0]).wait()
            pltpu.async_copy(
                v_hbm.at[idxs.at[pl.ds(_NBG * _BG, _BG)]], vb.at[0],
                gv[0]).wait()
            add_uv(0)
            pltpu.sync_copy(ub.at[0], ms_hbm.at[pl.ds(e0, _BG)])

    return gat(U, V, src, dst)


def _scatter_max(H, pk, cnts):
    d_out = H.shape[1]
    BH = 64
    CHP = 1024

    @functools.partial(
        pl.kernel,
        mesh=_mesh(),
        out_type=jax.ShapeDtypeStruct((_NP, d_out), jnp.float32),
        scratch_types=[
            pltpu.VMEM((_BKT + 1, d_out), jnp.float32),
            pltpu.VMEM((2, BH, d_out), jnp.float32),
            pltpu.VMEM((CHP + 16,), jnp.int32),
            pltpu.VMEM((2, BH), jnp.int32),
            pltpu.VMEM((16,), jnp.int32),
            pltpu.SemaphoreType.DMA,
            pltpu.SemaphoreType.DMA,
        ],
    )
    def scat(h_hbm, pk_hbm, cnt_hbm, out_hbm,
             tbl, hbuf, pkv, idxb, cntv, g0, g1):
        wid = _wid()
        gsem = [g0, g1]
        zero16 = jnp.zeros((16,), jnp.float32)

        def zb(i, _):
            for j in range(d_out // 16):
                tbl[i, pl.ds(j * 16, 16)] = zero16
            return 0

        lax.fori_loop(0, _BKT + 1, zb, 0)

        pltpu.sync_copy(cnt_hbm.at[pl.ds(pl.multiple_of(wid * 16, 16), 16)], cntv)
        n_pad = cntv[pl.ds(0, 16)][0]
        nfull = n_pad // CHP
        rb = lax.rem(n_pad, CHP) // BH

        def mk_idx(pkoff, slot):
            for j in range(BH // 16):
                idxb[slot, pl.ds(j * 16, 16)] = lax.shift_right_logical(
                    pkv[pl.ds(pkoff + j * 16, 16)], 9)

        def g_start(slot):
            pltpu.async_copy(h_hbm.at[idxb.at[slot]], hbuf.at[slot], gsem[slot])

        def g_wait(slot):
            pltpu.make_async_copy(
                h_hbm.at[idxb.at[slot]], hbuf.at[slot], gsem[slot]).wait()

        def fold(pkoff, slot):
            def eb(i, _):
                loc = pkv[pl.ds(pkoff + i, 16)][0] & 511
                for j in range(d_out // 16):
                    a = tbl[loc, pl.ds(j * 16, 16)]
                    b = hbuf[slot, i, pl.ds(j * 16, 16)]
                    tbl[loc, pl.ds(j * 16, 16)] = jnp.maximum(a, b)
                return 0

            lax.fori_loop(0, BH, eb, 0)

        def chunk(c, _):
            coff = pl.multiple_of(wid * _CAP, 8) + c * CHP
            pltpu.sync_copy(pk_hbm.at[pl.ds(coff, CHP)], pkv.at[pl.ds(0, CHP)])
            mk_idx(0, 0)
            g_start(0)
            for b in range(CHP // BH):
                slot = b % 2
                if b + 1 < CHP // BH:
                    mk_idx((b + 1) * BH, (b + 1) % 2)
                    g_start((b + 1) % 2)
                g_wait(slot)
                fold(b * BH, slot)
            return 0

        lax.fori_loop(0, nfull, chunk, 0)

        def rchunk(r, _):
            roff = pl.multiple_of(wid * _CAP, 8) + nfull * CHP + r * BH
            pltpu.sync_copy(pk_hbm.at[pl.ds(roff, BH)], pkv.at[pl.ds(0, BH)])
            mk_idx(0, 0)
            g_start(0)
            g_wait(0)
            fold(0, 0)
            return 0

        lax.fori_loop(0, rb, rchunk, 0)
        pltpu.sync_copy(tbl.at[pl.ds(0, _BKT)],
                        out_hbm.at[pl.ds(pl.multiple_of(wid * _BKT, _BKT), _BKT)])

    return scat(H, pk, cnts)


def _layer(xp, src, dst, pk, cnts, W1, b1, W2, b2):
    f = xp.shape[1]
    A = W1[:f] - W1[f:]
    B = W1[f:]
    U, V = _uv_matmul(xp, A, B, b1)
    ms = _gather_msum(U, V, src, dst)
    H = _edge_matmul(ms, W2, b2)
    return _scatter_max(H, pk, cnts)


@jax.jit
def kernel(x, edge_index, W1_1, b1_1, W2_1, b2_1, W1_2, b1_2, W2_2, b2_2,
           W1_3, b1_3, W2_3, b2_3, W1_4, b1_4, W2_4, b2_4):
    N = x.shape[0]
    src = edge_index[0]
    dst = edge_index[1]
    pk, cnts = _prep_sc(dst)
    xp = jnp.pad(x, ((0, _NP - N), (0, 0)))
    xp = _layer(xp, src, dst, pk, cnts, W1_1, b1_1, W2_1, b2_1)
    xp = _layer(xp, src, dst, pk, cnts, W1_2, b1_2, W2_2, b2_2)
    xp = _layer(xp, src, dst, pk, cnts, W1_3, b1_3, W2_3, b2_3)
    xp = _layer(xp, src, dst, pk, cnts, W1_4, b1_4, W2_4, b2_4)
    return xp[:N]
